# baseline (speedup 1.0000x reference)
import functools

import jax
import jax.numpy as jnp
from jax import lax
from jax.experimental import pallas as pl
from jax.experimental.pallas import tpu as pltpu
from jax.experimental.pallas import tpu_sc as plsc

_R = 512


_LN = 128


def _embed_gather(x_flat, table):
    n_tok = x_flat.shape[0]
    d = table.shape[1]
    seg = d // _LN
    idx2 = (x_flat.reshape(-1, 8)[:, None, :] * seg
            + jnp.arange(seg, dtype=jnp.int32)[None, :, None]).reshape(-1)
    table2 = table.reshape(-1, _LN)

    info = plsc.get_sparse_core_info()
    nc, ns = info.num_cores, info.num_subcores
    nw = nc * ns
    rows_out = n_tok * seg
    per_w = rows_out // nw
    k6 = per_w // _LN

    mesh = plsc.VectorSubcoreMesh(core_axis_name="c", subcore_axis_name="s")

    @functools.partial(
        pl.kernel,
        mesh=mesh,
        out_type=jax.ShapeDtypeStruct((rows_out, _LN), jnp.float32),
        scratch_types=[
            pltpu.VMEM((k6, _LN), jnp.int32),
            pltpu.VMEM((per_w, _LN), jnp.float32),
            pltpu.SemaphoreType.DMA,
        ],
    )
    def gather_kernel(idx_hbm, table_hbm, out_hbm, idx_v, rows_v, sem):
        wid = lax.axis_index("s") * nc + lax.axis_index("c")
        base = wid * per_w
        for b in range(k6):
            pltpu.sync_copy(idx_hbm.at[pl.ds(base + b * _LN, _LN)],
                            idx_v.at[b])
        copies = [
            pltpu.async_copy(table_hbm.at[idx_v.at[b]],
                             rows_v.at[pl.ds(b * _LN, _LN)], sem)
            for b in range(k6)
        ]
        for cp in copies:
            cp.wait()
        pltpu.sync_copy(rows_v, out_hbm.at[pl.ds(base, per_w)])

    return gather_kernel(idx2, table2)


_T0 = 16


def _two_level_scan(x, tp, rr, dd):
    t1 = rr // _T0
    x4 = x.reshape(t1, _T0, dd)
    s = 1
    while s < _T0:
        dk = tp[s - 1:s, :].reshape(1, 1, dd)
        zpad = jnp.zeros((t1, s, dd), jnp.float32)
        x4 = x4 + dk * jnp.concatenate([zpad, x4[:, :_T0 - s, :]], axis=1)
        s *= 2
    blk = x4[:, _T0 - 1, :]
    inc = blk
    s = 1
    while s < t1:
        dk = tp[s * _T0 - 1:s * _T0, :]
        zpad = jnp.zeros((s, dd), jnp.float32)
        inc = inc + dk * jnp.concatenate([zpad, inc[:t1 - s, :]], axis=0)
        s *= 2
    exc = jnp.concatenate([jnp.zeros((1, dd), jnp.float32), inc[:t1 - 1, :]],
                          axis=0)
    dsub = tp[:_T0, :].reshape(1, _T0, dd)
    x4 = x4 + dsub * exc.reshape(t1, 1, dd)
    return x4.reshape(rr, dd)


def _mega_body(cpb, cap, h_ref, td_ref, wr_ref, wk_ref, wv_ref, wo_ref,
               mk_ref, mv_ref, wc_ref, wd_ref, wout_ref, bout_ref,
               out_ref, cn_ref, cd_ref, tp_ref):
    i = pl.program_id(0)
    seg = h_ref.shape[1]
    rr = h_ref.shape[0] * h_ref.shape[2]
    dd = seg * h_ref.shape[3]

    @pl.when(i == 0)
    def _():
        e = jnp.exp(td_ref[...])
        tpos = lax.broadcasted_iota(jnp.int32, (rr, 1), 0).astype(jnp.float32)
        tp_ref[...] = jnp.exp(-(tpos + 1.0) * e)

    @pl.when(i % cpb == 0)
    def _():
        cn_ref[...] = jnp.zeros_like(cn_ref)
        cd_ref[...] = jnp.zeros_like(cd_ref)

    h4 = h_ref[...]
    h = jnp.concatenate(
        [h4[:, b, :, :].reshape(rr, _LN) for b in range(seg)], axis=1)
    tp = tp_ref[...]

    r = jax.nn.sigmoid(jnp.dot(h, wr_ref[...], preferred_element_type=jnp.float32))
    k = jnp.dot(h, wk_ref[...], preferred_element_type=jnp.float32)
    v = jnp.dot(h, wv_ref[...], preferred_element_type=jnp.float32)
    ek = jnp.exp(jnp.clip(k, -30.0, 30.0))

    num = _two_level_scan(ek * v, tp, rr, dd)
    den = _two_level_scan(ek, tp, rr, dd)

    num = num + tp * cn_ref[...]
    den = den + tp * cd_ref[...]
    cn_ref[...] = num[rr - 1:rr, :]
    cd_ref[...] = den[rr - 1:rr, :]
    wkv = num / (den + 1e-6)

    h2 = h + jnp.dot(r * wkv, wo_ref[...], preferred_element_type=jnp.float32)

    q = jnp.dot(h2, wc_ref[...], preferred_element_type=jnp.float32)
    c_dim = q.shape[1]
    scores = lax.dot_general(q, mk_ref[...], (((1,), (1,)), ((), ())),
                             preferred_element_type=jnp.float32)
    scores = scores * jnp.float32(1.0 / (c_dim ** 0.5))

    col = lax.broadcasted_iota(jnp.int32, (rr, cap), 1)
    m1 = jnp.max(scores, axis=1, keepdims=True)
    i1 = jnp.min(jnp.where(scores == m1, col, cap), axis=1, keepdims=True)
    masked = jnp.where(col == i1, jnp.float32(-jnp.inf), scores)
    m2 = jnp.max(masked, axis=1, keepdims=True)
    i2 = jnp.min(jnp.where(masked == m2, col, cap), axis=1, keepdims=True)
    e2 = jnp.exp(m2 - m1)
    w1 = 1.0 / (1.0 + e2)
    w2 = e2 / (1.0 + e2)
    wsel = jnp.where(col == i1, w1, 0.0) + jnp.where(col == i2, w2, 0.0)

    read = jnp.dot(wsel, mv_ref[...], preferred_element_type=jnp.float32)
    h3 = h2 + jnp.dot(read.astype(jnp.bfloat16), wd_ref[...],
                      preferred_element_type=jnp.float32)
    out_ref[...] = (jnp.dot(h3.astype(jnp.bfloat16), wout_ref[...],
                            preferred_element_type=jnp.float32)
                    + bout_ref[...])


def _tc_forward(h4, d, td, wr, wk, wv, wo, mem_keys, mem_values, wc, wd,
                w_out, b_out, chunks_per_batch):
    seg = d // _LN
    n_tok = h4.shape[0] // seg
    cap, c = mem_keys.shape
    vocab = w_out.shape[1]
    n_chunks = n_tok // _R
    h4 = h4.reshape(n_tok // 8, seg, 8, _LN)

    fixed4 = lambda i: (0, 0, 0, 0)
    fixed = lambda i: (0, 0)
    return pl.pallas_call(
        functools.partial(_mega_body, chunks_per_batch, cap),
        grid=(n_chunks,),
        in_specs=[
            pl.BlockSpec((_R // 8, seg, 8, _LN), lambda i: (i, 0, 0, 0)),
            pl.BlockSpec((1, d), fixed),
            pl.BlockSpec((d, d), fixed),
            pl.BlockSpec((d, d), fixed),
            pl.BlockSpec((d, d), fixed),
            pl.BlockSpec((d, d), fixed),
            pl.BlockSpec((cap, c), fixed),
            pl.BlockSpec((cap, c), fixed),
            pl.BlockSpec((d, c), fixed),
            pl.BlockSpec((c, d), fixed),
            pl.BlockSpec((d, vocab), fixed),
            pl.BlockSpec((1, vocab), fixed),
        ],
        out_specs=pl.BlockSpec((_R, vocab), lambda i: (i, 0)),
        out_shape=jax.ShapeDtypeStruct((n_tok, vocab), jnp.float32),
        scratch_shapes=[
            pltpu.VMEM((1, d), jnp.float32),
            pltpu.VMEM((1, d), jnp.float32),
            pltpu.VMEM((_R, d), jnp.float32),
        ],
        compiler_params=pltpu.CompilerParams(
            dimension_semantics=("arbitrary",),
        ),
    )(h4, td, wr, wk, wv, wo, mem_keys, mem_values, wc,
      wd.astype(jnp.bfloat16), w_out.astype(jnp.bfloat16), b_out)


def kernel(x, embed_table, time_decay, Wr, Wk, Wv, Wo, mem_keys, mem_values,
           Wc, Wd, W_out, b_out):
    b, s = x.shape
    vocab = W_out.shape[1]
    d = embed_table.shape[1]
    x_flat = x.reshape(-1).astype(jnp.int32)
    h4 = _embed_gather(x_flat, embed_table)
    out = _tc_forward(h4, d, time_decay.reshape(1, -1), Wr, Wk, Wv, Wo,
                      mem_keys, mem_values, Wc, Wd, W_out,
                      b_out.reshape(1, -1), chunks_per_batch=s // _R)
    return out.reshape(b, s, vocab)


# speedup vs baseline: 1.1217x; 1.1217x over previous
import functools

import jax
import jax.numpy as jnp
from jax import lax
from jax.experimental import pallas as pl
from jax.experimental.pallas import tpu as pltpu
from jax.experimental.pallas import tpu_sc as plsc

_R = 512


_LN = 128


def _embed_gather(x_flat, table):
    n_tok = x_flat.shape[0]
    d = table.shape[1]
    seg = d // _LN
    idx2 = (x_flat.reshape(-1, 8)[:, None, :] * seg
            + jnp.arange(seg, dtype=jnp.int32)[None, :, None]).reshape(-1)
    table2 = table.reshape(-1, _LN)

    info = plsc.get_sparse_core_info()
    nc, ns = info.num_cores, info.num_subcores
    nw = nc * ns
    rows_out = n_tok * seg
    per_w = rows_out // nw
    k6 = per_w // _LN

    mesh = plsc.VectorSubcoreMesh(core_axis_name="c", subcore_axis_name="s")

    @functools.partial(
        pl.kernel,
        mesh=mesh,
        out_type=jax.ShapeDtypeStruct((rows_out, _LN), jnp.float32),
        scratch_types=[
            pltpu.VMEM((k6, _LN), jnp.int32),
            pltpu.VMEM((per_w, _LN), jnp.float32),
            pltpu.SemaphoreType.DMA,
        ],
    )
    def gather_kernel(idx_hbm, table_hbm, out_hbm, idx_v, rows_v, sem):
        wid = lax.axis_index("s") * nc + lax.axis_index("c")
        base = wid * per_w
        for b in range(k6):
            pltpu.sync_copy(idx_hbm.at[pl.ds(base + b * _LN, _LN)],
                            idx_v.at[b])
        copies = [
            pltpu.async_copy(table_hbm.at[idx_v.at[b]],
                             rows_v.at[pl.ds(b * _LN, _LN)], sem)
            for b in range(k6)
        ]
        for cp in copies:
            cp.wait()
        pltpu.sync_copy(rows_v, out_hbm.at[pl.ds(base, per_w)])

    return gather_kernel(idx2, table2)


_T0 = 16


def _two_level_scan(x, tp, rr, dd):
    t1 = rr // _T0
    x4 = x.reshape(t1, _T0, dd)
    s = 1
    while s < _T0:
        dk = tp[s - 1:s, :].reshape(1, 1, dd)
        zpad = jnp.zeros((t1, s, dd), jnp.float32)
        x4 = x4 + dk * jnp.concatenate([zpad, x4[:, :_T0 - s, :]], axis=1)
        s *= 2
    blk = x4[:, _T0 - 1, :]
    inc = blk
    s = 1
    while s < t1:
        dk = tp[s * _T0 - 1:s * _T0, :]
        zpad = jnp.zeros((s, dd), jnp.float32)
        inc = inc + dk * jnp.concatenate([zpad, inc[:t1 - s, :]], axis=0)
        s *= 2
    exc = jnp.concatenate([jnp.zeros((1, dd), jnp.float32), inc[:t1 - 1, :]],
                          axis=0)
    dsub = tp[:_T0, :].reshape(1, _T0, dd)
    x4 = x4 + dsub * exc.reshape(t1, 1, dd)
    return x4.reshape(rr, dd)


def _mega_body(cpb, cap, h_ref, td_ref, wr_ref, wk_ref, wv_ref, wo_ref,
               mk_ref, mv_ref, wc_ref, wd_ref, wout_ref, bout_ref,
               out_ref, cn_ref, cd_ref, tp_ref):
    i = pl.program_id(0)
    seg = h_ref.shape[1]
    rr = h_ref.shape[0] * h_ref.shape[2]
    dd = seg * h_ref.shape[3]

    @pl.when(i == 0)
    def _():
        e = jnp.exp(td_ref[...])
        tpos = lax.broadcasted_iota(jnp.int32, (rr, 1), 0).astype(jnp.float32)
        tp_ref[...] = jnp.exp(-(tpos + 1.0) * e)

    @pl.when(i % cpb == 0)
    def _():
        cn_ref[...] = jnp.zeros_like(cn_ref)
        cd_ref[...] = jnp.zeros_like(cd_ref)

    h4 = h_ref[...]
    h = jnp.concatenate(
        [h4[:, b, :, :].reshape(rr, _LN) for b in range(seg)], axis=1)
    tp = tp_ref[...]

    r = jax.nn.sigmoid(jnp.dot(h, wr_ref[...], preferred_element_type=jnp.float32))
    k = jnp.dot(h, wk_ref[...], preferred_element_type=jnp.float32)
    v = jnp.dot(h, wv_ref[...], preferred_element_type=jnp.float32)
    ek = jnp.exp(jnp.clip(k, -30.0, 30.0))

    num = _two_level_scan(ek * v, tp, rr, dd)
    den = _two_level_scan(ek, tp, rr, dd)

    num = num + tp * cn_ref[...]
    den = den + tp * cd_ref[...]
    cn_ref[...] = num[rr - 1:rr, :]
    cd_ref[...] = den[rr - 1:rr, :]
    wkv = num / (den + 1e-6)

    h2 = h + jnp.dot(r * wkv, wo_ref[...], preferred_element_type=jnp.float32)

    q = jnp.dot(h2, wc_ref[...], preferred_element_type=jnp.float32)
    c_dim = q.shape[1]
    scores = lax.dot_general(q, mk_ref[...], (((1,), (1,)), ((), ())),
                             preferred_element_type=jnp.float32)
    scores = scores * jnp.float32(1.0 / (c_dim ** 0.5))

    col = lax.broadcasted_iota(jnp.int32, (rr, cap), 1)
    m1 = jnp.max(scores, axis=1, keepdims=True)
    i1 = jnp.min(jnp.where(scores == m1, col, cap), axis=1, keepdims=True)
    masked = jnp.where(col == i1, jnp.float32(-jnp.inf), scores)
    m2 = jnp.max(masked, axis=1, keepdims=True)
    i2 = jnp.min(jnp.where(masked == m2, col, cap), axis=1, keepdims=True)
    e2 = jnp.exp(m2 - m1)
    w1 = 1.0 / (1.0 + e2)
    w2 = e2 / (1.0 + e2)
    wsel = jnp.where(col == i1, w1, 0.0) + jnp.where(col == i2, w2, 0.0)

    read = jnp.dot(wsel, mv_ref[...], preferred_element_type=jnp.float32)
    h3 = h2 + jnp.dot(read.astype(jnp.bfloat16), wd_ref[...],
                      preferred_element_type=jnp.float32)
    outt = lax.dot_general(wout_ref[...], h3.astype(jnp.bfloat16),
                           (((1,), (1,)), ((), ())),
                           preferred_element_type=jnp.float32)
    out_ref[...] = (outt + bout_ref[...]).reshape(1, outt.shape[0],
                                                  outt.shape[1])


def _tc_forward(h4, d, td, wr, wk, wv, wo, mem_keys, mem_values, wc, wd,
                w_out, b_out, chunks_per_batch):
    seg = d // _LN
    n_tok = h4.shape[0] // seg
    cap, c = mem_keys.shape
    vocab = w_out.shape[1]
    n_chunks = n_tok // _R
    h4 = h4.reshape(n_tok // 8, seg, 8, _LN)

    fixed4 = lambda i: (0, 0, 0, 0)
    fixed = lambda i: (0, 0)
    return pl.pallas_call(
        functools.partial(_mega_body, chunks_per_batch, cap),
        grid=(n_chunks,),
        in_specs=[
            pl.BlockSpec((_R // 8, seg, 8, _LN), lambda i: (i, 0, 0, 0)),
            pl.BlockSpec((1, d), fixed),
            pl.BlockSpec((d, d), fixed),
            pl.BlockSpec((d, d), fixed),
            pl.BlockSpec((d, d), fixed),
            pl.BlockSpec((d, d), fixed),
            pl.BlockSpec((cap, c), fixed),
            pl.BlockSpec((cap, c), fixed),
            pl.BlockSpec((d, c), fixed),
            pl.BlockSpec((c, d), fixed),
            pl.BlockSpec((vocab, d), fixed),
            pl.BlockSpec((vocab, 1), fixed),
        ],
        out_specs=pl.BlockSpec(
            (1, vocab, _R),
            lambda i: (i // chunks_per_batch, 0, i % chunks_per_batch)),
        out_shape=jax.ShapeDtypeStruct(
            (n_tok // (chunks_per_batch * _R), vocab,
             chunks_per_batch * _R), jnp.float32),
        scratch_shapes=[
            pltpu.VMEM((1, d), jnp.float32),
            pltpu.VMEM((1, d), jnp.float32),
            pltpu.VMEM((_R, d), jnp.float32),
        ],
        compiler_params=pltpu.CompilerParams(
            dimension_semantics=("arbitrary",),
        ),
    )(h4, td, wr, wk, wv, wo, mem_keys, mem_values, wc,
      wd.astype(jnp.bfloat16), w_out.T.astype(jnp.bfloat16),
      b_out.reshape(-1, 1))


def kernel(x, embed_table, time_decay, Wr, Wk, Wv, Wo, mem_keys, mem_values,
           Wc, Wd, W_out, b_out):
    b, s = x.shape
    vocab = W_out.shape[1]
    d = embed_table.shape[1]
    x_flat = x.reshape(-1).astype(jnp.int32)
    h4 = _embed_gather(x_flat, embed_table)
    out = _tc_forward(h4, d, time_decay.reshape(1, -1), Wr, Wk, Wv, Wo,
                      mem_keys, mem_values, Wc, Wd, W_out,
                      b_out, chunks_per_batch=s // _R)
    return out.swapaxes(1, 2)
